# baseline (speedup 1.0000x reference)
import functools

import jax
import jax.numpy as jnp
from jax.experimental import pallas as pl
from jax.experimental.pallas import tpu as pltpu


def _ceil_to(x, m):
    return ((x + m - 1) // m) * m


def _largest_divisor(n, cap):
    for t in range(min(n, cap), 0, -1):
        if n % t == 0:
            return t
    return 1


def _proj_kernel(y_ref, c_ref, u_ref, abspart_ref):
    T = y_ref.shape[2]
    yblk = y_ref[...]
    c = c_ref[...]
    for t in range(T):
        u_ref[t] = jnp.dot(yblk[:, 0, t, :], c,
                           preferred_element_type=jnp.float32)
    ab = jnp.sum(jnp.abs(yblk), axis=(1, 2))
    abspart_ref[0] = jnp.sum(ab, keepdims=True)


def _powers_kernel(a_ref, c_ref, u_ref, g_ref, mtm_ref, rhs_ref):
    s = u_ref.shape[0]
    h = a_ref.shape[0]
    a = a_ref[...]
    c = c_ref[...]
    S = jax.lax.dot_general(c, c, dimension_numbers=(((0,), (0,)), ((), ())),
                            preferred_element_type=jnp.float32)
    rows = jax.lax.broadcasted_iota(jnp.int32, (h, h), 0)
    cols = jax.lax.broadcasted_iota(jnp.int32, (h, h), 1)
    oc = (rows == cols).astype(jnp.float32)
    mtm = jnp.zeros((h, h), jnp.float32)
    rhs = jnp.zeros((u_ref.shape[1], h), jnp.float32)
    for k in range(s):
        g_ref[:, k * h:(k + 1) * h] = oc
        sg = jnp.dot(S, oc, preferred_element_type=jnp.float32)
        mtm = mtm + jax.lax.dot_general(
            oc, sg, dimension_numbers=(((0,), (0,)), ((), ())),
            preferred_element_type=jnp.float32)
        rhs = rhs + jnp.dot(u_ref[k], oc, preferred_element_type=jnp.float32)
        oc = jnp.dot(oc, a, preferred_element_type=jnp.float32)
    mtm_ref[...] = mtm
    rhs_ref[...] = rhs


def _emit_kernel(y_ref, c_ref, g_ref, x0_ref, invlam_ref,
                 o_ref, yhat_ref, w_ref, sqpart_ref):
    T = y_ref.shape[2]
    h = x0_ref.shape[0]
    c = c_ref[...]
    x0 = x0_ref[...]
    inv_lam = invlam_ref[0]
    ores = jnp.dot(c, g_ref[...], preferred_element_type=jnp.float32)
    yblk = y_ref[...]
    sq = jnp.zeros((1, 1), jnp.float32)
    for t in range(T):
        o_t = ores[:, t * h:(t + 1) * h]
        o_ref[t] = o_t
        yh = jnp.dot(o_t, x0, preferred_element_type=jnp.float32)
        yhat_ref[t] = yh
        ycol = yblk[:, 0, t, :].T
        wv = (ycol - yh) * inv_lam
        w_ref[t] = wv
        sq = sq + jnp.sum(wv * wv, keepdims=True)
    sqpart_ref[0] = sq


@jax.jit
def _forward(x, C, A):
    b, s, c, h, w = x.shape
    obs = c * h * w
    hidden = C.shape[1]
    M = float(obs)
    prediction_alpha = 1.0

    obs_p = _ceil_to(obs, 128)
    hid_p = _ceil_to(hidden, 128)
    b_p = _ceil_to(b, 8)
    T1 = _largest_divisor(s, 8)
    T3 = _largest_divisor(s, 2)

    y = x.reshape(b, s, obs).astype(jnp.float32)
    logdet = jnp.zeros((b * s,), jnp.float32)
    Y = y.reshape(b, s * obs)

    if (b_p, obs_p) != (b, obs):
        y_nat = jnp.zeros((b_p, s, obs_p), jnp.float32).at[:b, :, :obs].set(y)
    else:
        y_nat = y
    if (obs_p, hid_p) != (obs, hidden):
        C_pad = jnp.zeros((obs_p, hid_p), jnp.float32).at[:obs, :hidden].set(C)
    else:
        C_pad = C
    if hid_p != hidden:
        A_pad = jnp.zeros((hid_p, hid_p), jnp.float32).at[:hidden, :hidden].set(A)
    else:
        A_pad = A

    n1 = s // T1
    y4_1 = y_nat.reshape(b_p, n1, T1, obs_p)
    u, abspart = pl.pallas_call(
        _proj_kernel,
        out_shape=(
            jax.ShapeDtypeStruct((s, b_p, hid_p), jnp.float32),
            jax.ShapeDtypeStruct((n1, 1, 1), jnp.float32),
        ),
        grid_spec=pltpu.PrefetchScalarGridSpec(
            num_scalar_prefetch=0,
            grid=(n1,),
            in_specs=[
                pl.BlockSpec((b_p, 1, T1, obs_p), lambda i: (0, i, 0, 0)),
                pl.BlockSpec((obs_p, hid_p), lambda i: (0, 0)),
            ],
            out_specs=[
                pl.BlockSpec((T1, b_p, hid_p), lambda i: (i, 0, 0)),
                pl.BlockSpec((1, 1, 1), lambda i: (i, 0, 0)),
            ],
        ),
        compiler_params=pltpu.CompilerParams(
            dimension_semantics=("parallel",),
            vmem_limit_bytes=48 * 1024 * 1024),
    )(y4_1, C_pad)

    G, MtM_pad, rhs_pad = pl.pallas_call(
        _powers_kernel,
        out_shape=(
            jax.ShapeDtypeStruct((hid_p, s * hid_p), jnp.float32),
            jax.ShapeDtypeStruct((hid_p, hid_p), jnp.float32),
            jax.ShapeDtypeStruct((b_p, hid_p), jnp.float32),
        ),
        grid_spec=pltpu.PrefetchScalarGridSpec(
            num_scalar_prefetch=0,
            grid=(1,),
            in_specs=[
                pl.BlockSpec((hid_p, hid_p), lambda i: (0, 0)),
                pl.BlockSpec((obs_p, hid_p), lambda i: (0, 0)),
                pl.BlockSpec((s, b_p, hid_p), lambda i: (0, 0, 0)),
            ],
            out_specs=[
                pl.BlockSpec((hid_p, s * hid_p), lambda i: (0, 0)),
                pl.BlockSpec((hid_p, hid_p), lambda i: (0, 0)),
                pl.BlockSpec((b_p, hid_p), lambda i: (0, 0)),
            ],
        ),
        compiler_params=pltpu.CompilerParams(
            dimension_semantics=("arbitrary",),
            vmem_limit_bytes=48 * 1024 * 1024),
    )(A_pad, C_pad, u)

    abssum = jnp.sum(abspart)
    scaling_lambda = abssum / (b * s * obs)

    MtM = MtM_pad[:hidden, :hidden]
    rhs = rhs_pad[:b, :hidden]
    L = jnp.linalg.cholesky(MtM)
    z = jax.scipy.linalg.solve_triangular(L, rhs.T, lower=True)
    x0 = jax.scipy.linalg.solve_triangular(L.T, z, lower=False)

    if (b_p, hid_p) != (b, hidden):
        x0_pad = jnp.zeros((hid_p, b_p), jnp.float32).at[:hidden, :b].set(x0)
    else:
        x0_pad = x0
    inv_lambda = (1.0 / scaling_lambda).reshape(1).astype(jnp.float32)

    n3 = s // T3
    y4_3 = y_nat.reshape(b_p, n3, T3, obs_p)
    O_pad, Yhat_col, W_col, sqpart = pl.pallas_call(
        _emit_kernel,
        out_shape=(
            jax.ShapeDtypeStruct((s, obs_p, hid_p), jnp.float32),
            jax.ShapeDtypeStruct((s, obs_p, b_p), jnp.float32),
            jax.ShapeDtypeStruct((s, obs_p, b_p), jnp.float32),
            jax.ShapeDtypeStruct((n3, 1, 1), jnp.float32),
        ),
        grid_spec=pltpu.PrefetchScalarGridSpec(
            num_scalar_prefetch=0,
            grid=(n3,),
            in_specs=[
                pl.BlockSpec((b_p, 1, T3, obs_p), lambda i: (0, i, 0, 0)),
                pl.BlockSpec((obs_p, hid_p), lambda i: (0, 0)),
                pl.BlockSpec((hid_p, T3 * hid_p), lambda i: (0, i)),
                pl.BlockSpec((hid_p, b_p), lambda i: (0, 0)),
                pl.BlockSpec(memory_space=pltpu.MemorySpace.SMEM),
            ],
            out_specs=[
                pl.BlockSpec((T3, obs_p, hid_p), lambda i: (i, 0, 0)),
                pl.BlockSpec((T3, obs_p, b_p), lambda i: (i, 0, 0)),
                pl.BlockSpec((T3, obs_p, b_p), lambda i: (i, 0, 0)),
                pl.BlockSpec((1, 1, 1), lambda i: (i, 0, 0)),
            ],
        ),
        compiler_params=pltpu.CompilerParams(
            dimension_semantics=("parallel",),
            vmem_limit_bytes=48 * 1024 * 1024),
    )(y4_3, C_pad, G, x0_pad, inv_lambda)

    prediction_error = jnp.sum(sqpart) / (s * obs * b)
    log_likelihood = jnp.mean(logdet / M) - jnp.log(scaling_lambda)
    loss = -log_likelihood + prediction_alpha * prediction_error

    if (obs_p, hid_p) != (obs, hidden):
        O = O_pad[:, :obs, :hidden].reshape(s * obs, hidden)
    else:
        O = O_pad.reshape(s * obs, hidden)
    if (obs_p, b_p) != (obs, b):
        Yhat = Yhat_col[:, :obs, :b].reshape(s * obs, b)
        W = W_col[:, :obs, :b].reshape(s * obs, b)
    else:
        Yhat = Yhat_col.reshape(s * obs, b)
        W = W_col.reshape(s * obs, b)
    return loss, (W, Y, Yhat, y, x0, logdet / M, prediction_error, O,
                  scaling_lambda)


def kernel(x, C, A):
    return _forward(x, C, A)


# speedup vs baseline: 1.2458x; 1.2458x over previous
import functools

import jax
import jax.numpy as jnp
from jax.experimental import pallas as pl
from jax.experimental.pallas import tpu as pltpu


def _ceil_to(x, m):
    return ((x + m - 1) // m) * m


def _largest_divisor(n, cap):
    for t in range(min(n, cap), 0, -1):
        if n % t == 0:
            return t
    return 1


def _proj_kernel(y_ref, c_ref, u_ref, yflat_ref, abspart_ref):
    T = y_ref.shape[2]
    obs = y_ref.shape[3]
    yblk = y_ref[...]
    c = c_ref[...]
    for t in range(T):
        y_t = yblk[:, 0, t, :]
        u_ref[t] = jnp.dot(y_t, c, preferred_element_type=jnp.float32)
        yflat_ref[:, t * obs:(t + 1) * obs] = y_t
    ab = jnp.sum(jnp.abs(yblk), axis=(1, 2))
    abspart_ref[0] = jnp.sum(ab, keepdims=True)


def _powers_kernel(a_ref, c_ref, u_ref, g_ref, mtm_ref, rhs_ref):
    s = u_ref.shape[0]
    h = a_ref.shape[0]
    a = a_ref[...]
    c = c_ref[...]
    S = jax.lax.dot_general(c, c, dimension_numbers=(((0,), (0,)), ((), ())),
                            preferred_element_type=jnp.float32)
    rows = jax.lax.broadcasted_iota(jnp.int32, (h, h), 0)
    cols = jax.lax.broadcasted_iota(jnp.int32, (h, h), 1)
    oc = (rows == cols).astype(jnp.float32)
    mtm = jnp.zeros((h, h), jnp.float32)
    rhs = jnp.zeros((u_ref.shape[1], h), jnp.float32)
    for k in range(s):
        g_ref[:, k * h:(k + 1) * h] = oc
        sg = jnp.dot(S, oc, preferred_element_type=jnp.float32)
        mtm = mtm + jax.lax.dot_general(
            oc, sg, dimension_numbers=(((0,), (0,)), ((), ())),
            preferred_element_type=jnp.float32)
        rhs = rhs + jnp.dot(u_ref[k], oc, preferred_element_type=jnp.float32)
        oc = jnp.dot(oc, a, preferred_element_type=jnp.float32)
    mtm_ref[...] = mtm
    rhs_ref[...] = rhs


def _emit_kernel(y_ref, c_ref, g_ref, x0t_ref, invlam_ref,
                 o_ref, yhat_ref, w_ref, sqpart_ref, xs_ref):
    T = y_ref.shape[2]
    obs = y_ref.shape[3]
    bp = x0t_ref.shape[0]
    h = x0t_ref.shape[1]
    c = c_ref[...]
    x0t = x0t_ref[...]
    inv_lam = invlam_ref[0]
    ores = jnp.dot(c, g_ref[...], preferred_element_type=jnp.float32)
    yblk = y_ref[...]
    for t in range(T):
        o_ref[t * obs:(t + 1) * obs, :] = ores[:, t * h:(t + 1) * h]
        xs_ref[t * bp:(t + 1) * bp, :] = jax.lax.dot_general(
            x0t, g_ref[:, t * h:(t + 1) * h],
            dimension_numbers=(((1,), (1,)), ((), ())),
            preferred_element_type=jnp.float32)
    yh = jax.lax.dot_general(
        xs_ref[...], c, dimension_numbers=(((1,), (1,)), ((), ())),
        preferred_element_type=jnp.float32)
    sq = jnp.zeros((1, 1), jnp.float32)
    for t in range(T):
        yh_t = yh[t * bp:(t + 1) * bp, :]
        yhat_ref[t] = yh_t
        wv = (yblk[:, 0, t, :] - yh_t) * inv_lam
        w_ref[t] = wv
        sq = sq + jnp.sum(wv * wv, keepdims=True)
    sqpart_ref[0] = sq


@jax.jit
def _forward(x, C, A):
    b, s, c, h, w = x.shape
    obs = c * h * w
    hidden = C.shape[1]
    M = float(obs)
    prediction_alpha = 1.0

    obs_p = _ceil_to(obs, 128)
    hid_p = _ceil_to(hidden, 128)
    b_p = _ceil_to(b, 8)
    T1 = _largest_divisor(s, 8)
    T3 = _largest_divisor(s, 4)

    y = x.reshape(b, s, obs).astype(jnp.float32)
    logdet = jnp.zeros((b * s,), jnp.float32)

    if (b_p, obs_p) != (b, obs):
        y_nat = jnp.zeros((b_p, s, obs_p), jnp.float32).at[:b, :, :obs].set(y)
    else:
        y_nat = y
    if (obs_p, hid_p) != (obs, hidden):
        C_pad = jnp.zeros((obs_p, hid_p), jnp.float32).at[:obs, :hidden].set(C)
    else:
        C_pad = C
    if hid_p != hidden:
        A_pad = jnp.zeros((hid_p, hid_p), jnp.float32).at[:hidden, :hidden].set(A)
    else:
        A_pad = A

    n1 = s // T1
    y4_1 = y_nat.reshape(b_p, n1, T1, obs_p)
    u, Yflat, abspart = pl.pallas_call(
        _proj_kernel,
        out_shape=(
            jax.ShapeDtypeStruct((s, b_p, hid_p), jnp.float32),
            jax.ShapeDtypeStruct((b_p, s * obs_p), jnp.float32),
            jax.ShapeDtypeStruct((n1, 1, 1), jnp.float32),
        ),
        grid_spec=pltpu.PrefetchScalarGridSpec(
            num_scalar_prefetch=0,
            grid=(n1,),
            in_specs=[
                pl.BlockSpec((b_p, 1, T1, obs_p), lambda i: (0, i, 0, 0)),
                pl.BlockSpec((obs_p, hid_p), lambda i: (0, 0)),
            ],
            out_specs=[
                pl.BlockSpec((T1, b_p, hid_p), lambda i: (i, 0, 0)),
                pl.BlockSpec((b_p, T1 * obs_p), lambda i: (0, i)),
                pl.BlockSpec((1, 1, 1), lambda i: (i, 0, 0)),
            ],
        ),
        compiler_params=pltpu.CompilerParams(
            dimension_semantics=("parallel",),
            vmem_limit_bytes=48 * 1024 * 1024),
    )(y4_1, C_pad)

    G, MtM_pad, rhs_pad = pl.pallas_call(
        _powers_kernel,
        out_shape=(
            jax.ShapeDtypeStruct((hid_p, s * hid_p), jnp.float32),
            jax.ShapeDtypeStruct((hid_p, hid_p), jnp.float32),
            jax.ShapeDtypeStruct((b_p, hid_p), jnp.float32),
        ),
        grid_spec=pltpu.PrefetchScalarGridSpec(
            num_scalar_prefetch=0,
            grid=(1,),
            in_specs=[
                pl.BlockSpec((hid_p, hid_p), lambda i: (0, 0)),
                pl.BlockSpec((obs_p, hid_p), lambda i: (0, 0)),
                pl.BlockSpec((s, b_p, hid_p), lambda i: (0, 0, 0)),
            ],
            out_specs=[
                pl.BlockSpec((hid_p, s * hid_p), lambda i: (0, 0)),
                pl.BlockSpec((hid_p, hid_p), lambda i: (0, 0)),
                pl.BlockSpec((b_p, hid_p), lambda i: (0, 0)),
            ],
        ),
        compiler_params=pltpu.CompilerParams(
            dimension_semantics=("arbitrary",),
            vmem_limit_bytes=48 * 1024 * 1024),
    )(A_pad, C_pad, u)

    abssum = jnp.sum(abspart)
    scaling_lambda = abssum / (b * s * obs)

    MtM = MtM_pad[:hidden, :hidden]
    rhs = rhs_pad[:b, :hidden]
    L = jnp.linalg.cholesky(MtM)
    z = jax.scipy.linalg.solve_triangular(L, rhs.T, lower=True)
    x0 = jax.scipy.linalg.solve_triangular(L.T, z, lower=False)

    if (b_p, hid_p) != (b, hidden):
        x0t_pad = jnp.zeros((b_p, hid_p), jnp.float32).at[:b, :hidden].set(x0.T)
    else:
        x0t_pad = x0.T
    inv_lambda = (1.0 / scaling_lambda).reshape(1).astype(jnp.float32)

    n3 = s // T3
    y4_3 = y_nat.reshape(b_p, n3, T3, obs_p)
    O_flat, Yhat_pad, W_pad, sqpart = pl.pallas_call(
        _emit_kernel,
        out_shape=(
            jax.ShapeDtypeStruct((s * obs_p, hid_p), jnp.float32),
            jax.ShapeDtypeStruct((s, b_p, obs_p), jnp.float32),
            jax.ShapeDtypeStruct((s, b_p, obs_p), jnp.float32),
            jax.ShapeDtypeStruct((n3, 1, 1), jnp.float32),
        ),
        grid_spec=pltpu.PrefetchScalarGridSpec(
            num_scalar_prefetch=0,
            grid=(n3,),
            in_specs=[
                pl.BlockSpec((b_p, 1, T3, obs_p), lambda i: (0, i, 0, 0)),
                pl.BlockSpec((obs_p, hid_p), lambda i: (0, 0)),
                pl.BlockSpec((hid_p, T3 * hid_p), lambda i: (0, i)),
                pl.BlockSpec((b_p, hid_p), lambda i: (0, 0)),
                pl.BlockSpec(memory_space=pltpu.MemorySpace.SMEM),
            ],
            out_specs=[
                pl.BlockSpec((T3 * obs_p, hid_p), lambda i: (i, 0)),
                pl.BlockSpec((T3, b_p, obs_p), lambda i: (i, 0, 0)),
                pl.BlockSpec((T3, b_p, obs_p), lambda i: (i, 0, 0)),
                pl.BlockSpec((1, 1, 1), lambda i: (i, 0, 0)),
            ],
            scratch_shapes=[
                pltpu.VMEM((T3 * b_p, hid_p), jnp.float32),
            ],
        ),
        compiler_params=pltpu.CompilerParams(
            dimension_semantics=("parallel",),
            vmem_limit_bytes=48 * 1024 * 1024),
    )(y4_3, C_pad, G, x0t_pad, inv_lambda)

    prediction_error = jnp.sum(sqpart) / (s * obs * b)
    log_likelihood = jnp.mean(logdet / M) - jnp.log(scaling_lambda)
    loss = -log_likelihood + prediction_alpha * prediction_error

    if (b_p, obs_p) != (b, obs):
        Y = Yflat.reshape(b_p, s, obs_p)[:b, :, :obs].reshape(b, s * obs)
    else:
        Y = Yflat
    if (obs_p, hid_p) != (obs, hidden):
        O = O_flat.reshape(s, obs_p, hid_p)[:, :obs, :hidden].reshape(
            s * obs, hidden)
    else:
        O = O_flat
    Yhat = jnp.transpose(Yhat_pad[:, :b, :obs], (0, 2, 1)).reshape(s * obs, b)
    W = jnp.transpose(W_pad[:, :b, :obs], (0, 2, 1)).reshape(s * obs, b)
    return loss, (W, Y, Yhat, y, x0, logdet / M, prediction_error, O,
                  scaling_lambda)


def kernel(x, C, A):
    return _forward(x, C, A)


# speedup vs baseline: 1.7673x; 1.4187x over previous
import functools

import jax
import jax.numpy as jnp
from jax.experimental import pallas as pl
from jax.experimental.pallas import tpu as pltpu


def _ceil_to(x, m):
    return ((x + m - 1) // m) * m


def _largest_divisor(n, cap):
    for t in range(min(n, cap), 0, -1):
        if n % t == 0:
            return t
    return 1


def _proj_kernel(y_ref, c_ref, u_ref, abspart_ref):
    T = y_ref.shape[0]
    bp = y_ref.shape[1]
    h = c_ref.shape[1]
    yblk = y_ref[...]
    yflat = yblk.reshape(T * bp, y_ref.shape[2])
    U = jnp.dot(yflat, c_ref[...], preferred_element_type=jnp.float32)
    for t in range(T):
        u_ref[:, t * h:(t + 1) * h] = U[t * bp:(t + 1) * bp, :]
    abspart_ref[0] = jnp.sum(jnp.abs(yflat), keepdims=True)


def _make_powers_kernel(s, doubling):
    def _powers_kernel(a_ref, c_ref, u_ref, g_ref, mtm_ref, rhs_ref):
        h = a_ref.shape[0]
        c = c_ref[...]
        S = jax.lax.dot_general(
            c, c, dimension_numbers=(((0,), (0,)), ((), ())),
            preferred_element_type=jnp.float32)
        rows = jax.lax.broadcasted_iota(jnp.int32, (h, h), 0)
        cols = jax.lax.broadcasted_iota(jnp.int32, (h, h), 1)
        eye = (rows == cols).astype(jnp.float32)
        if doubling:
            g_ref[:, 0:h] = eye
            gpow = a_ref[...]
            mtm = S
            n = 1
            while n < s:
                g_ref[:, n * h:2 * n * h] = jnp.dot(
                    gpow, g_ref[:, 0:n * h],
                    preferred_element_type=jnp.float32)
                mg = jnp.dot(mtm, gpow, preferred_element_type=jnp.float32)
                mtm = mtm + jax.lax.dot_general(
                    gpow, mg, dimension_numbers=(((0,), (0,)), ((), ())),
                    preferred_element_type=jnp.float32)
                n *= 2
                if n < s:
                    gpow = jnp.dot(gpow, gpow,
                                   preferred_element_type=jnp.float32)
        else:
            a = a_ref[...]
            oc = eye
            mtm = jnp.zeros((h, h), jnp.float32)
            for k in range(s):
                g_ref[:, k * h:(k + 1) * h] = oc
                sg = jnp.dot(S, oc, preferred_element_type=jnp.float32)
                mtm = mtm + jax.lax.dot_general(
                    oc, sg, dimension_numbers=(((0,), (0,)), ((), ())),
                    preferred_element_type=jnp.float32)
                oc = jnp.dot(oc, a, preferred_element_type=jnp.float32)
        rhs = jnp.zeros((u_ref.shape[0], h), jnp.float32)
        for k in range(s):
            rhs = rhs + jnp.dot(u_ref[:, k * h:(k + 1) * h],
                                g_ref[:, k * h:(k + 1) * h],
                                preferred_element_type=jnp.float32)
        mtm_ref[...] = mtm
        rhs_ref[...] = rhs
    return _powers_kernel


def _emit_kernel(y_ref, c_ref, g_ref, x0t_ref, invlam_ref,
                 o_ref, yhat_ref, w_ref, sqpart_ref, xs_ref):
    T = y_ref.shape[0]
    bp = y_ref.shape[1]
    h = x0t_ref.shape[1]
    c = c_ref[...]
    x0t = x0t_ref[...]
    inv_lam = invlam_ref[0]
    ores = jnp.dot(c, g_ref[...], preferred_element_type=jnp.float32)
    for t in range(T):
        o_ref[t] = ores[:, t * h:(t + 1) * h]
        xs_ref[t * bp:(t + 1) * bp, :] = jax.lax.dot_general(
            x0t, g_ref[:, t * h:(t + 1) * h],
            dimension_numbers=(((1,), (1,)), ((), ())),
            preferred_element_type=jnp.float32)
    yh = jax.lax.dot_general(
        xs_ref[...], c, dimension_numbers=(((1,), (1,)), ((), ())),
        preferred_element_type=jnp.float32)
    yblk = y_ref[...]
    yflat = yblk.reshape(T * bp, y_ref.shape[2])
    w = (yflat - yh) * inv_lam
    yhat_ref[...] = yh.reshape(T, bp, y_ref.shape[2])
    w_ref[...] = w.reshape(T, bp, y_ref.shape[2])
    sqpart_ref[0] = jnp.sum(w * w, keepdims=True)


@jax.jit
def _forward(x, C, A):
    b, s, c, h, w = x.shape
    obs = c * h * w
    hidden = C.shape[1]
    M = float(obs)
    prediction_alpha = 1.0

    obs_p = _ceil_to(obs, 128)
    hid_p = _ceil_to(hidden, 128)
    b_p = _ceil_to(b, 8)
    T1 = _largest_divisor(s, 8)
    T3 = _largest_divisor(s, 4)

    y = x.reshape(b, s, obs).astype(jnp.float32)
    logdet = jnp.zeros((b * s,), jnp.float32)
    Y = y.reshape(b, s * obs)

    y_sbo = jnp.transpose(y, (1, 0, 2))
    if (b_p, obs_p) != (b, obs):
        y_pad = jnp.zeros((s, b_p, obs_p), jnp.float32).at[:, :b, :obs].set(y_sbo)
    else:
        y_pad = y_sbo
    if (obs_p, hid_p) != (obs, hidden):
        C_pad = jnp.zeros((obs_p, hid_p), jnp.float32).at[:obs, :hidden].set(C)
    else:
        C_pad = C
    if hid_p != hidden:
        A_pad = jnp.zeros((hid_p, hid_p), jnp.float32).at[:hidden, :hidden].set(A)
    else:
        A_pad = A

    n1 = s // T1
    u, abspart = pl.pallas_call(
        _proj_kernel,
        out_shape=(
            jax.ShapeDtypeStruct((b_p, s * hid_p), jnp.float32),
            jax.ShapeDtypeStruct((n1, 1, 1), jnp.float32),
        ),
        grid_spec=pltpu.PrefetchScalarGridSpec(
            num_scalar_prefetch=0,
            grid=(n1,),
            in_specs=[
                pl.BlockSpec((T1, b_p, obs_p), lambda i: (i, 0, 0)),
                pl.BlockSpec((obs_p, hid_p), lambda i: (0, 0)),
            ],
            out_specs=[
                pl.BlockSpec((b_p, T1 * hid_p), lambda i: (0, i)),
                pl.BlockSpec((1, 1, 1), lambda i: (i, 0, 0)),
            ],
        ),
        compiler_params=pltpu.CompilerParams(
            dimension_semantics=("parallel",),
            vmem_limit_bytes=48 * 1024 * 1024),
    )(y_pad, C_pad)

    is_pow2 = (s & (s - 1)) == 0
    G, MtM_pad, rhs_pad = pl.pallas_call(
        _make_powers_kernel(s, is_pow2),
        out_shape=(
            jax.ShapeDtypeStruct((hid_p, s * hid_p), jnp.float32),
            jax.ShapeDtypeStruct((hid_p, hid_p), jnp.float32),
            jax.ShapeDtypeStruct((b_p, hid_p), jnp.float32),
        ),
        grid_spec=pltpu.PrefetchScalarGridSpec(
            num_scalar_prefetch=0,
            grid=(1,),
            in_specs=[
                pl.BlockSpec((hid_p, hid_p), lambda i: (0, 0)),
                pl.BlockSpec((obs_p, hid_p), lambda i: (0, 0)),
                pl.BlockSpec((b_p, s * hid_p), lambda i: (0, 0)),
            ],
            out_specs=[
                pl.BlockSpec((hid_p, s * hid_p), lambda i: (0, 0)),
                pl.BlockSpec((hid_p, hid_p), lambda i: (0, 0)),
                pl.BlockSpec((b_p, hid_p), lambda i: (0, 0)),
            ],
        ),
        compiler_params=pltpu.CompilerParams(
            dimension_semantics=("arbitrary",),
            vmem_limit_bytes=48 * 1024 * 1024),
    )(A_pad, C_pad, u)

    abssum = jnp.sum(abspart)
    scaling_lambda = abssum / (b * s * obs)

    MtM = MtM_pad[:hidden, :hidden]
    rhs = rhs_pad[:b, :hidden]
    L = jnp.linalg.cholesky(MtM)
    z = jax.scipy.linalg.solve_triangular(L, rhs.T, lower=True)
    x0 = jax.scipy.linalg.solve_triangular(L.T, z, lower=False)

    if (b_p, hid_p) != (b, hidden):
        x0t_pad = jnp.zeros((b_p, hid_p), jnp.float32).at[:b, :hidden].set(x0.T)
    else:
        x0t_pad = x0.T
    inv_lambda = (1.0 / scaling_lambda).reshape(1).astype(jnp.float32)

    n3 = s // T3
    O_pad, Yhat_pad, W_pad, sqpart = pl.pallas_call(
        _emit_kernel,
        out_shape=(
            jax.ShapeDtypeStruct((s, obs_p, hid_p), jnp.float32),
            jax.ShapeDtypeStruct((s, b_p, obs_p), jnp.float32),
            jax.ShapeDtypeStruct((s, b_p, obs_p), jnp.float32),
            jax.ShapeDtypeStruct((n3, 1, 1), jnp.float32),
        ),
        grid_spec=pltpu.PrefetchScalarGridSpec(
            num_scalar_prefetch=0,
            grid=(n3,),
            in_specs=[
                pl.BlockSpec((T3, b_p, obs_p), lambda i: (i, 0, 0)),
                pl.BlockSpec((obs_p, hid_p), lambda i: (0, 0)),
                pl.BlockSpec((hid_p, T3 * hid_p), lambda i: (0, i)),
                pl.BlockSpec((b_p, hid_p), lambda i: (0, 0)),
                pl.BlockSpec(memory_space=pltpu.MemorySpace.SMEM),
            ],
            out_specs=[
                pl.BlockSpec((T3, obs_p, hid_p), lambda i: (i, 0, 0)),
                pl.BlockSpec((T3, b_p, obs_p), lambda i: (i, 0, 0)),
                pl.BlockSpec((T3, b_p, obs_p), lambda i: (i, 0, 0)),
                pl.BlockSpec((1, 1, 1), lambda i: (i, 0, 0)),
            ],
            scratch_shapes=[
                pltpu.VMEM((T3 * b_p, hid_p), jnp.float32),
            ],
        ),
        compiler_params=pltpu.CompilerParams(
            dimension_semantics=("parallel",),
            vmem_limit_bytes=48 * 1024 * 1024),
    )(y_pad, C_pad, G, x0t_pad, inv_lambda)

    prediction_error = jnp.sum(sqpart) / (s * obs * b)
    log_likelihood = jnp.mean(logdet / M) - jnp.log(scaling_lambda)
    loss = -log_likelihood + prediction_alpha * prediction_error

    O = O_pad[:, :obs, :hidden].reshape(s * obs, hidden)
    Yhat = jnp.transpose(Yhat_pad[:, :b, :obs], (0, 2, 1)).reshape(s * obs, b)
    W = jnp.transpose(W_pad[:, :b, :obs], (0, 2, 1)).reshape(s * obs, b)
    return loss, (W, Y, Yhat, y, x0, logdet / M, prediction_error, O,
                  scaling_lambda)


def kernel(x, C, A):
    return _forward(x, C, A)


# speedup vs baseline: 1.8831x; 1.0655x over previous
import functools

import jax
import jax.numpy as jnp
from jax.experimental import pallas as pl
from jax.experimental.pallas import tpu as pltpu


def _ceil_to(x, m):
    return ((x + m - 1) // m) * m


def _largest_divisor(n, cap):
    for t in range(min(n, cap), 0, -1):
        if n % t == 0:
            return t
    return 1


def _proj_kernel(y_ref, c_ref, u_ref, abspart_ref):
    T = y_ref.shape[0]
    bp = y_ref.shape[1]
    h = c_ref.shape[1]
    yblk = y_ref[...]
    yflat = yblk.reshape(T * bp, y_ref.shape[2])
    U = jnp.dot(yflat, c_ref[...], preferred_element_type=jnp.float32)
    for t in range(T):
        u_ref[:, t * h:(t + 1) * h] = U[t * bp:(t + 1) * bp, :]
    abspart_ref[0] = jnp.sum(jnp.abs(yflat), keepdims=True)


def _make_powers_kernel(s, doubling, hidden, denom):
    def _powers_kernel(a_ref, c_ref, u_ref, abs_ref, g_ref, x0t_ref,
                       invlam_ref):
        h = a_ref.shape[0]
        c = c_ref[...]
        S = jax.lax.dot_general(
            c, c, dimension_numbers=(((0,), (0,)), ((), ())),
            preferred_element_type=jnp.float32)
        rows = jax.lax.broadcasted_iota(jnp.int32, (h, h), 0)
        cols = jax.lax.broadcasted_iota(jnp.int32, (h, h), 1)
        eye = (rows == cols).astype(jnp.float32)
        if doubling:
            g_ref[:, 0:h] = eye
            gpow = a_ref[...]
            mtm = S
            n = 1
            while n < s:
                g_ref[:, n * h:2 * n * h] = jnp.dot(
                    gpow, g_ref[:, 0:n * h],
                    preferred_element_type=jnp.float32)
                mg = jnp.dot(mtm, gpow, preferred_element_type=jnp.float32)
                mtm = mtm + jax.lax.dot_general(
                    gpow, mg, dimension_numbers=(((0,), (0,)), ((), ())),
                    preferred_element_type=jnp.float32)
                n *= 2
                if n < s:
                    gpow = jnp.dot(gpow, gpow,
                                   preferred_element_type=jnp.float32)
        else:
            a = a_ref[...]
            oc = eye
            mtm = jnp.zeros((h, h), jnp.float32)
            for k in range(s):
                g_ref[:, k * h:(k + 1) * h] = oc
                sg = jnp.dot(S, oc, preferred_element_type=jnp.float32)
                mtm = mtm + jax.lax.dot_general(
                    oc, sg, dimension_numbers=(((0,), (0,)), ((), ())),
                    preferred_element_type=jnp.float32)
                oc = jnp.dot(oc, a, preferred_element_type=jnp.float32)
        rhs = jnp.zeros((u_ref.shape[0], h), jnp.float32)
        for k in range(s):
            rhs = rhs + jnp.dot(u_ref[:, k * h:(k + 1) * h],
                                g_ref[:, k * h:(k + 1) * h],
                                preferred_element_type=jnp.float32)
        if hidden != h:
            pad_eye = jnp.logical_and(rows == cols, rows >= hidden)
            m = mtm + pad_eye.astype(jnp.float32)
        else:
            m = mtm
        ninf = jnp.max(jnp.sum(jnp.abs(m), axis=1, keepdims=True),
                       keepdims=True)
        xinv = eye * (1.0 / ninf)
        two_eye = eye + eye
        for _ in range(8):
            mx = jnp.dot(m, xinv, preferred_element_type=jnp.float32)
            xinv = jnp.dot(xinv, two_eye - mx,
                           preferred_element_type=jnp.float32)
        x0t_ref[...] = jnp.dot(rhs, xinv, preferred_element_type=jnp.float32)
        lam = jnp.sum(abs_ref[...], axis=0) * (1.0 / denom)
        invlam_ref[...] = 1.0 / lam
    return _powers_kernel


def _emit_kernel(y_ref, c_ref, g_ref, x0t_ref, invlam_ref,
                 o_ref, yhat_ref, w_ref, sqpart_ref, xs_ref):
    T = y_ref.shape[0]
    bp = y_ref.shape[1]
    h = x0t_ref.shape[1]
    c = c_ref[...]
    x0t = x0t_ref[...]
    inv_lam = invlam_ref[0]
    ores = jnp.dot(c, g_ref[...], preferred_element_type=jnp.float32)
    for t in range(T):
        o_ref[t] = ores[:, t * h:(t + 1) * h]
        xs_ref[t * bp:(t + 1) * bp, :] = jax.lax.dot_general(
            x0t, g_ref[:, t * h:(t + 1) * h],
            dimension_numbers=(((1,), (1,)), ((), ())),
            preferred_element_type=jnp.float32)
    yh = jax.lax.dot_general(
        xs_ref[...], c, dimension_numbers=(((1,), (1,)), ((), ())),
        preferred_element_type=jnp.float32)
    yblk = y_ref[...]
    yflat = yblk.reshape(T * bp, y_ref.shape[2])
    w = (yflat - yh) * inv_lam
    yhat_ref[...] = yh.reshape(T, bp, y_ref.shape[2])
    w_ref[...] = w.reshape(T, bp, y_ref.shape[2])
    sqpart_ref[0] = jnp.sum(w * w, keepdims=True)


@jax.jit
def _forward(x, C, A):
    b, s, c, h, w = x.shape
    obs = c * h * w
    hidden = C.shape[1]
    M = float(obs)
    prediction_alpha = 1.0

    obs_p = _ceil_to(obs, 128)
    hid_p = _ceil_to(hidden, 128)
    b_p = _ceil_to(b, 8)
    T1 = _largest_divisor(s, 8)
    T3 = _largest_divisor(s, 4)

    y = x.reshape(b, s, obs).astype(jnp.float32)
    logdet = jnp.zeros((b * s,), jnp.float32)
    Y = y.reshape(b, s * obs)

    y_sbo = jnp.transpose(y, (1, 0, 2))
    if (b_p, obs_p) != (b, obs):
        y_pad = jnp.zeros((s, b_p, obs_p), jnp.float32).at[:, :b, :obs].set(y_sbo)
    else:
        y_pad = y_sbo
    if (obs_p, hid_p) != (obs, hidden):
        C_pad = jnp.zeros((obs_p, hid_p), jnp.float32).at[:obs, :hidden].set(C)
    else:
        C_pad = C
    if hid_p != hidden:
        A_pad = jnp.zeros((hid_p, hid_p), jnp.float32).at[:hidden, :hidden].set(A)
    else:
        A_pad = A

    n1 = s // T1
    u, abspart = pl.pallas_call(
        _proj_kernel,
        out_shape=(
            jax.ShapeDtypeStruct((b_p, s * hid_p), jnp.float32),
            jax.ShapeDtypeStruct((n1, 1, 1), jnp.float32),
        ),
        grid_spec=pltpu.PrefetchScalarGridSpec(
            num_scalar_prefetch=0,
            grid=(n1,),
            in_specs=[
                pl.BlockSpec((T1, b_p, obs_p), lambda i: (i, 0, 0)),
                pl.BlockSpec((obs_p, hid_p), lambda i: (0, 0)),
            ],
            out_specs=[
                pl.BlockSpec((b_p, T1 * hid_p), lambda i: (0, i)),
                pl.BlockSpec((1, 1, 1), lambda i: (i, 0, 0)),
            ],
        ),
        compiler_params=pltpu.CompilerParams(
            dimension_semantics=("parallel",),
            vmem_limit_bytes=48 * 1024 * 1024),
    )(y_pad, C_pad)

    is_pow2 = (s & (s - 1)) == 0
    G, x0t_pad, invlam2 = pl.pallas_call(
        _make_powers_kernel(s, is_pow2, hidden, float(b * s * obs)),
        out_shape=(
            jax.ShapeDtypeStruct((hid_p, s * hid_p), jnp.float32),
            jax.ShapeDtypeStruct((b_p, hid_p), jnp.float32),
            jax.ShapeDtypeStruct((1, 1), jnp.float32),
        ),
        grid_spec=pltpu.PrefetchScalarGridSpec(
            num_scalar_prefetch=0,
            grid=(1,),
            in_specs=[
                pl.BlockSpec((hid_p, hid_p), lambda i: (0, 0)),
                pl.BlockSpec((obs_p, hid_p), lambda i: (0, 0)),
                pl.BlockSpec((b_p, s * hid_p), lambda i: (0, 0)),
                pl.BlockSpec((n1, 1, 1), lambda i: (0, 0, 0)),
            ],
            out_specs=[
                pl.BlockSpec((hid_p, s * hid_p), lambda i: (0, 0)),
                pl.BlockSpec((b_p, hid_p), lambda i: (0, 0)),
                pl.BlockSpec((1, 1), lambda i: (0, 0)),
            ],
        ),
        compiler_params=pltpu.CompilerParams(
            dimension_semantics=("arbitrary",),
            vmem_limit_bytes=48 * 1024 * 1024),
    )(A_pad, C_pad, u, abspart)

    scaling_lambda = jnp.sum(abspart) / (b * s * obs)
    x0 = x0t_pad[:b, :hidden].T
    inv_lambda = invlam2.reshape(1)

    n3 = s // T3
    O_pad, Yhat_pad, W_pad, sqpart = pl.pallas_call(
        _emit_kernel,
        out_shape=(
            jax.ShapeDtypeStruct((s, obs_p, hid_p), jnp.float32),
            jax.ShapeDtypeStruct((s, b_p, obs_p), jnp.float32),
            jax.ShapeDtypeStruct((s, b_p, obs_p), jnp.float32),
            jax.ShapeDtypeStruct((n3, 1, 1), jnp.float32),
        ),
        grid_spec=pltpu.PrefetchScalarGridSpec(
            num_scalar_prefetch=0,
            grid=(n3,),
            in_specs=[
                pl.BlockSpec((T3, b_p, obs_p), lambda i: (i, 0, 0)),
                pl.BlockSpec((obs_p, hid_p), lambda i: (0, 0)),
                pl.BlockSpec((hid_p, T3 * hid_p), lambda i: (0, i)),
                pl.BlockSpec((b_p, hid_p), lambda i: (0, 0)),
                pl.BlockSpec(memory_space=pltpu.MemorySpace.SMEM),
            ],
            out_specs=[
                pl.BlockSpec((T3, obs_p, hid_p), lambda i: (i, 0, 0)),
                pl.BlockSpec((T3, b_p, obs_p), lambda i: (i, 0, 0)),
                pl.BlockSpec((T3, b_p, obs_p), lambda i: (i, 0, 0)),
                pl.BlockSpec((1, 1, 1), lambda i: (i, 0, 0)),
            ],
            scratch_shapes=[
                pltpu.VMEM((T3 * b_p, hid_p), jnp.float32),
            ],
        ),
        compiler_params=pltpu.CompilerParams(
            dimension_semantics=("parallel",),
            vmem_limit_bytes=48 * 1024 * 1024),
    )(y_pad, C_pad, G, x0t_pad, inv_lambda)

    prediction_error = jnp.sum(sqpart) / (s * obs * b)
    log_likelihood = jnp.mean(logdet / M) - jnp.log(scaling_lambda)
    loss = -log_likelihood + prediction_alpha * prediction_error

    O = O_pad[:, :obs, :hidden].reshape(s * obs, hidden)
    Yhat = jnp.transpose(Yhat_pad[:, :b, :obs], (0, 2, 1)).reshape(s * obs, b)
    W = jnp.transpose(W_pad[:, :b, :obs], (0, 2, 1)).reshape(s * obs, b)
    return loss, (W, Y, Yhat, y, x0, logdet / M, prediction_error, O,
                  scaling_lambda)


def kernel(x, C, A):
    return _forward(x, C, A)
